# baseline (speedup 1.0000x reference)
import jax
import jax.numpy as jnp
from jax import lax
from jax.experimental import pallas as pl
from jax.experimental.pallas import tpu as pltpu
from jax.experimental.pallas import tpu_sc as plsc

VOCAB = 1000000
B = 16384
S = 50
SF = 56
SP = 64
D = 64
DP = 128
NC, NS = 2, 16
NW = NC * NS
B_PER_W = B // NW
GB = 4
N_GROUPS = B_PER_W // GB


def _emb_body(idx_hbm, table_hbm, out_hbm, idx_v, rows0, rows1,
              gsem0, gsem1, wsem0, wsem1):
    wid = lax.axis_index("s") * NC + lax.axis_index("c")
    b_base = wid * B_PER_W
    rows = (rows0, rows1)
    gsem = (gsem0, gsem1)
    wsem = (wsem0, wsem1)

    pltpu.sync_copy(idx_hbm.at[pl.ds(b_base * SP, B_PER_W * SP)], idx_v)

    def fire_gathers(g, b):
        for j in range(GB):
            idx_off = pl.multiple_of((g * GB + j) * SP, 8)
            pltpu.async_copy(
                table_hbm.at[idx_v.at[pl.ds(idx_off, SF)]],
                rows[b].at[j],
                gsem[b],
            )

    def drain_gathers(b):
        pltpu.make_async_copy(
            out_hbm.at[pl.ds(0, GB), :, :], rows[b], gsem[b]
        ).wait()

    def fire_write(g, b):
        b_off = pl.multiple_of(b_base + g * GB, 4)
        pltpu.async_copy(
            rows[b], out_hbm.at[pl.ds(b_off, GB), :, :], wsem[b]
        )

    def drain_write(b):
        pltpu.make_async_copy(
            rows[b], out_hbm.at[pl.ds(0, GB), :, :], wsem[b]
        ).wait()

    fire_gathers(0, 0)
    fire_gathers(1, 1)

    def step(i, _):
        t = 2 * i
        drain_gathers(0)
        fire_write(t, 0)
        drain_gathers(1)
        fire_write(t + 1, 1)
        drain_write(0)
        fire_gathers(t + 2, 0)
        drain_write(1)
        fire_gathers(t + 3, 1)
        return 0

    lax.fori_loop(0, (N_GROUPS - 2) // 2, step, 0)

    drain_gathers(0)
    fire_write(N_GROUPS - 2, 0)
    drain_gathers(1)
    fire_write(N_GROUPS - 1, 1)
    drain_write(0)
    drain_write(1)


@jax.jit
def _emb(xp_flat, table128):
    mesh = plsc.VectorSubcoreMesh(core_axis_name="c", subcore_axis_name="s")
    run = pl.kernel(
        _emb_body,
        mesh=mesh,
        out_type=jax.ShapeDtypeStruct((B, SF, DP), jnp.float32),
        scratch_types=[
            pltpu.VMEM((B_PER_W * SP,), jnp.int32),
            pltpu.VMEM((GB, SF, DP), jnp.float32),
            pltpu.VMEM((GB, SF, DP), jnp.float32),
            pltpu.SemaphoreType.DMA,
            pltpu.SemaphoreType.DMA,
            pltpu.SemaphoreType.DMA,
            pltpu.SemaphoreType.DMA,
        ],
    )
    return run(xp_flat, table128)


def kernel(x, weight):
    xp = jnp.pad(x.astype(jnp.int32), ((0, 0), (0, SP - S)))
    w128 = jnp.pad(weight, ((0, 0), (0, DP - D)))
    o = _emb(xp.reshape(-1), w128)
    return o[:, :S, :D]


# speedup vs baseline: 1.0029x; 1.0029x over previous
import jax
import jax.numpy as jnp
from jax import lax
from jax.experimental import pallas as pl
from jax.experimental.pallas import tpu as pltpu
from jax.experimental.pallas import tpu_sc as plsc

VOCAB = 1000000
B = 16384
S = 50
SF = 56
D = 64
DP = 128
NC, NS = 2, 16
NW = NC * NS
R_TOTAL = B * SF
R_PER_W = R_TOTAL // NW
K = 128
G = 2
ROWS_PER_GROUP = K * G
N_GROUPS = R_PER_W // ROWS_PER_GROUP


def _emb_body(idx_hbm, table_hbm, out_hbm, idx_v, rows0, rows1,
              gsem0, gsem1, wsem0, wsem1):
    wid = lax.axis_index("s") * NC + lax.axis_index("c")
    base = wid * R_PER_W
    rows = (rows0, rows1)
    gsem = (gsem0, gsem1)
    wsem = (wsem0, wsem1)

    pltpu.sync_copy(idx_hbm.at[pl.ds(base, R_PER_W)], idx_v)

    def fire_gathers(g, b):
        row_base = g * ROWS_PER_GROUP
        for j in range(G):
            pltpu.async_copy(
                table_hbm.at[idx_v.at[pl.ds(row_base + j * K, K)]],
                rows[b].at[pl.ds(j * K, K), :],
                gsem[b],
            )

    def drain_gathers(b):
        pltpu.make_async_copy(
            out_hbm.at[pl.ds(0, ROWS_PER_GROUP), :], rows[b], gsem[b]
        ).wait()

    def fire_write(g, b):
        out_off = pl.multiple_of(base + g * ROWS_PER_GROUP, 8)
        pltpu.async_copy(
            rows[b], out_hbm.at[pl.ds(out_off, ROWS_PER_GROUP), :], wsem[b]
        )

    def drain_write(b):
        pltpu.make_async_copy(
            rows[b], out_hbm.at[pl.ds(0, ROWS_PER_GROUP), :], wsem[b]
        ).wait()

    fire_gathers(0, 0)
    fire_gathers(1, 1)

    def step(i, _):
        t = 2 * i
        drain_gathers(0)
        fire_write(t, 0)
        drain_gathers(1)
        fire_write(t + 1, 1)
        drain_write(0)
        fire_gathers(t + 2, 0)
        drain_write(1)
        fire_gathers(t + 3, 1)
        return 0

    lax.fori_loop(0, (N_GROUPS - 2) // 2, step, 0)

    drain_gathers(0)
    fire_write(N_GROUPS - 2, 0)
    drain_gathers(1)
    fire_write(N_GROUPS - 1, 1)
    drain_write(0)
    drain_write(1)


@jax.jit
def _emb(xf_flat, table128):
    mesh = plsc.VectorSubcoreMesh(core_axis_name="c", subcore_axis_name="s")
    run = pl.kernel(
        _emb_body,
        mesh=mesh,
        out_type=jax.ShapeDtypeStruct((R_TOTAL, DP), jnp.float32),
        scratch_types=[
            pltpu.VMEM((R_PER_W,), jnp.int32),
            pltpu.VMEM((ROWS_PER_GROUP, DP), jnp.float32),
            pltpu.VMEM((ROWS_PER_GROUP, DP), jnp.float32),
            pltpu.SemaphoreType.DMA,
            pltpu.SemaphoreType.DMA,
            pltpu.SemaphoreType.DMA,
            pltpu.SemaphoreType.DMA,
        ],
    )
    return run(xf_flat, table128)


def kernel(x, weight):
    xf = jnp.pad(x.astype(jnp.int32), ((0, 0), (0, SF - S)))
    w128 = jnp.pad(weight, ((0, 0), (0, DP - D)))
    o = _emb(xf.reshape(-1), w128)
    return o.reshape(B, SF, DP)[:, :S, :D]
